# Initial kernel scaffold; baseline (speedup 1.0000x reference)
import jax
import jax.numpy as jnp
from jax.experimental import pallas as pl


def kernel(x, edge_index, edge_attr, W_f, b_f, W_s, b_s, W1, b1, W2, b2):
    raise NotImplementedError("write your pallas kernel here")


# speedup vs baseline: 1.7159x; 1.7159x over previous
import functools

import jax
import jax.numpy as jnp
from jax import lax
from jax.experimental import pallas as pl
from jax.experimental.pallas import tpu as pltpu
from jax.experimental.pallas import tpu_sc as plsc

_NC = 2
_NS = 16
_LANES = 16


def _proj_body(x_ref, wi_ref, wj_ref, bi_ref, ti_ref, tj_ref):
    xb = x_ref[...]
    ti_ref[...] = (
        jnp.dot(xb, wi_ref[...], preferred_element_type=jnp.float32)
        + bi_ref[...]
    )
    tj_ref[...] = jnp.dot(xb, wj_ref[...], preferred_element_type=jnp.float32)


def _edge_body(g_ref, ea_ref, we_ref, m_ref):
    c = m_ref.shape[1]
    z = g_ref[...] + jnp.dot(
        ea_ref[...], we_ref[...], preferred_element_type=jnp.float32
    )
    a = z[:, :c]
    s = z[:, c:]
    sig = 1.0 / (1.0 + jnp.exp(-a))
    sp = jnp.maximum(s, 0.0) + jnp.log(1.0 + jnp.exp(-jnp.abs(s)))
    m_ref[...] = sig * sp


def _mlp_body(o_ref, w1_ref, b1_ref, w2_ref, b2_ref, y_ref):
    h = jnp.maximum(
        jnp.dot(o_ref[...], w1_ref[...], preferred_element_type=jnp.float32)
        + b1_ref[...],
        0.0,
    )
    y_ref[...] = (
        jnp.dot(h, w2_ref[...], preferred_element_type=jnp.float32)
        + b2_ref[...]
    )


@functools.lru_cache(maxsize=None)
def _make_gather(E, H, KG):
    NW = _NC * _NS
    epw = E // NW
    nchunk = epw // KG
    mesh = plsc.VectorSubcoreMesh(core_axis_name="c", subcore_axis_name="s")

    @functools.partial(
        pl.kernel,
        out_type=jax.ShapeDtypeStruct((E, H), jnp.float32),
        mesh=mesh,
        scratch_types=[
            pltpu.VMEM((KG,), jnp.int32),
            pltpu.VMEM((KG,), jnp.int32),
            pltpu.VMEM((KG, H), jnp.float32),
            pltpu.VMEM((KG, H), jnp.float32),
            pltpu.SemaphoreType.DMA,
            pltpu.SemaphoreType.DMA,
        ],
    )
    def gather_k(ti_hbm, tj_hbm, dst_hbm, src_hbm, g_hbm,
                 idx_d, idx_s, buf_i, buf_j, sem_i, sem_j):
        wid = lax.axis_index("s") * _NC + lax.axis_index("c")
        base = wid * epw

        def chunk_body(k, carry):
            e0 = base + k * KG
            pltpu.sync_copy(dst_hbm.at[pl.ds(e0, KG)], idx_d)
            pltpu.sync_copy(src_hbm.at[pl.ds(e0, KG)], idx_s)
            cp_i = pltpu.async_copy(ti_hbm.at[idx_d], buf_i, sem_i)
            cp_j = pltpu.async_copy(tj_hbm.at[idx_s], buf_j, sem_j)
            cp_i.wait()
            cp_j.wait()

            def row_body(r, c2):
                for cc in range(H // _LANES):
                    sl = pl.ds(cc * _LANES, _LANES)
                    buf_i[r, sl] = buf_i[r, sl] + buf_j[r, sl]
                return c2

            lax.fori_loop(0, KG, row_body, 0)
            pltpu.sync_copy(buf_i, g_hbm.at[pl.ds(e0, KG), :])
            return carry

        lax.fori_loop(0, nchunk, chunk_body, 0)

    return gather_k


@functools.lru_cache(maxsize=None)
def _make_scatter(E, N, C, CE):
    CS = C // _NC
    ept = E // _NS
    nchunk = ept // CE
    rpt = (N // _NS) // 8 * 8
    tail = N - rpt * _NS
    mesh = plsc.VectorSubcoreMesh(core_axis_name="c", subcore_axis_name="s")

    @functools.partial(
        pl.kernel,
        out_type=jax.ShapeDtypeStruct((N, C), jnp.float32),
        mesh=mesh,
        scratch_types=[
            pltpu.VMEM_SHARED((N, CS), jnp.float32),
            pltpu.VMEM((CE, CS), jnp.float32),
            pltpu.VMEM((CE,), jnp.int32),
        ],
    )
    def scatter_k(m_hbm, dst_hbm, x_hbm, out_hbm, acc_sh, buf, idxb):
        c = lax.axis_index("c")
        s = lax.axis_index("s")
        col0 = c * CS
        r0 = s * rpt
        pltpu.sync_copy(
            x_hbm.at[pl.ds(r0, rpt), pl.ds(col0, CS)],
            acc_sh.at[pl.ds(r0, rpt), :],
        )
        if tail:
            @pl.when(s == _NS - 1)
            def _init_tail():
                pltpu.sync_copy(
                    x_hbm.at[pl.ds(rpt * _NS, tail), pl.ds(col0, CS)],
                    acc_sh.at[pl.ds(rpt * _NS, tail), :],
                )
        plsc.subcore_barrier()

        def chunk_body(k, carry):
            e0 = s * ept + k * CE
            pltpu.sync_copy(dst_hbm.at[pl.ds(e0, CE)], idxb)
            pltpu.sync_copy(m_hbm.at[pl.ds(e0, CE), pl.ds(col0, CS)], buf)
            pltpu.sync_copy(buf, acc_sh.at[idxb], add=True)
            return carry

        lax.fori_loop(0, nchunk, chunk_body, 0)
        plsc.subcore_barrier()
        pltpu.sync_copy(
            acc_sh.at[pl.ds(r0, rpt), :],
            out_hbm.at[pl.ds(r0, rpt), pl.ds(col0, CS)],
        )
        if tail:
            @pl.when(s == _NS - 1)
            def _write_tail():
                pltpu.sync_copy(
                    acc_sh.at[pl.ds(rpt * _NS, tail), :],
                    out_hbm.at[pl.ds(rpt * _NS, tail), pl.ds(col0, CS)],
                )

    return scatter_k


def kernel(x, edge_index, edge_attr, W_f, b_f, W_s, b_s, W1, b1, W2, b2):
    N, C = x.shape
    E, D_E = edge_attr.shape
    H = 2 * C

    src = edge_index[0].astype(jnp.int32)
    dst = edge_index[1].astype(jnp.int32)
    W_i = jnp.concatenate([W_f[:C], W_s[:C]], axis=1)
    W_j = jnp.concatenate([W_f[C:2 * C], W_s[C:2 * C]], axis=1)
    W_e = jnp.concatenate([W_f[2 * C:], W_s[2 * C:]], axis=1)
    b_cat = jnp.concatenate([b_f, b_s])[None, :]

    BN = 1000
    t_i, t_j = pl.pallas_call(
        _proj_body,
        grid=(N // BN,),
        in_specs=[
            pl.BlockSpec((BN, C), lambda i: (i, 0)),
            pl.BlockSpec((C, H), lambda i: (0, 0)),
            pl.BlockSpec((C, H), lambda i: (0, 0)),
            pl.BlockSpec((1, H), lambda i: (0, 0)),
        ],
        out_specs=[pl.BlockSpec((BN, H), lambda i: (i, 0))] * 2,
        out_shape=[jax.ShapeDtypeStruct((N, H), jnp.float32)] * 2,
    )(x, W_i, W_j, b_cat)

    g = _make_gather(E, H, 40)(t_i, t_j, dst, src)

    BE = 2000
    m = pl.pallas_call(
        _edge_body,
        grid=(E // BE,),
        in_specs=[
            pl.BlockSpec((BE, H), lambda i: (i, 0)),
            pl.BlockSpec((BE, D_E), lambda i: (i, 0)),
            pl.BlockSpec((D_E, H), lambda i: (0, 0)),
        ],
        out_specs=pl.BlockSpec((BE, C), lambda i: (i, 0)),
        out_shape=jax.ShapeDtypeStruct((E, C), jnp.float32),
    )(g, edge_attr, W_e)

    onode = _make_scatter(E, N, C, 80)(m, dst, x)

    y = pl.pallas_call(
        _mlp_body,
        grid=(N // BN,),
        in_specs=[
            pl.BlockSpec((BN, C), lambda i: (i, 0)),
            pl.BlockSpec((C, C), lambda i: (0, 0)),
            pl.BlockSpec((1, C), lambda i: (0, 0)),
            pl.BlockSpec((C, C), lambda i: (0, 0)),
            pl.BlockSpec((1, C), lambda i: (0, 0)),
        ],
        out_specs=pl.BlockSpec((BN, C), lambda i: (i, 0)),
        out_shape=jax.ShapeDtypeStruct((N, C), jnp.float32),
    )(onode, W1, b1[None], W2, b2[None])
    return y


# speedup vs baseline: 2.2546x; 1.3140x over previous
import functools

import jax
import jax.numpy as jnp
from jax import lax
from jax.experimental import pallas as pl
from jax.experimental.pallas import tpu as pltpu
from jax.experimental.pallas import tpu_sc as plsc

_NC = 2
_NS = 16
_LANES = 16


def _proj_body(x_ref, wi_ref, wj_ref, bi_ref, ti_ref, tj_ref):
    xb = x_ref[...]
    ti_ref[...] = (
        jnp.dot(xb, wi_ref[...], preferred_element_type=jnp.float32)
        + bi_ref[...]
    )
    tj_ref[...] = jnp.dot(xb, wj_ref[...], preferred_element_type=jnp.float32)


def _edge_body(g_ref, ea_ref, we_ref, m_ref):
    c = m_ref.shape[1]
    z = g_ref[...] + jnp.dot(
        ea_ref[...], we_ref[...], preferred_element_type=jnp.float32
    )
    a = z[:, :c]
    s = z[:, c:]
    sig = 1.0 / (1.0 + jnp.exp(-a))
    sp = jnp.maximum(s, 0.0) + jnp.log(1.0 + jnp.exp(-jnp.abs(s)))
    m_ref[...] = sig * sp


def _mlp_body(o_ref, w1_ref, b1_ref, w2_ref, b2_ref, y_ref):
    h = jnp.maximum(
        jnp.dot(o_ref[...], w1_ref[...], preferred_element_type=jnp.float32)
        + b1_ref[...],
        0.0,
    )
    y_ref[...] = (
        jnp.dot(h, w2_ref[...], preferred_element_type=jnp.float32)
        + b2_ref[...]
    )


@functools.lru_cache(maxsize=None)
def _make_gather(E, H, KG):
    NW = _NC * _NS
    epw = E // NW
    nchunk = epw // KG
    npair = nchunk // 2
    mesh = plsc.VectorSubcoreMesh(core_axis_name="c", subcore_axis_name="s")

    @functools.partial(
        pl.kernel,
        out_type=jax.ShapeDtypeStruct((E, H), jnp.float32),
        mesh=mesh,
        scratch_types=[
            pltpu.VMEM((epw,), jnp.int32),
            pltpu.VMEM((epw,), jnp.int32),
            pltpu.VMEM((KG, H), jnp.float32),
            pltpu.VMEM((KG, H), jnp.float32),
            pltpu.VMEM((KG, H), jnp.float32),
            pltpu.VMEM((KG, H), jnp.float32),
            pltpu.SemaphoreType.DMA,
            pltpu.SemaphoreType.DMA,
            pltpu.SemaphoreType.DMA,
            pltpu.SemaphoreType.DMA,
            pltpu.SemaphoreType.DMA,
            pltpu.SemaphoreType.DMA,
        ],
    )
    def gather_k(ti_hbm, tj_hbm, dst_hbm, src_hbm, g_hbm,
                 idx_d, idx_s, bi0, bj0, bi1, bj1,
                 si0, sj0, si1, sj1, ws0, ws1):
        wid = lax.axis_index("s") * _NC + lax.axis_index("c")
        base = wid * epw
        pltpu.sync_copy(dst_hbm.at[pl.ds(base, epw)], idx_d)
        pltpu.sync_copy(src_hbm.at[pl.ds(base, epw)], idx_s)

        def _start(k, bi, bj, si, sj):
            sl = pl.ds(k * KG, KG)
            return (
                pltpu.async_copy(ti_hbm.at[idx_d.at[sl]], bi, si),
                pltpu.async_copy(tj_hbm.at[idx_s.at[sl]], bj, sj),
            )

        def _add(bi, bj):
            def row_body(r, c2):
                for cc in range(H // _LANES):
                    sl = pl.ds(cc * _LANES, _LANES)
                    plsc.addupdate(bi.at[r, sl], bj[r, sl])
                return c2

            lax.fori_loop(0, KG, row_body, 0)

        def pair_body(p, carry):
            c0 = 2 * p
            c1 = c0 + 1
            di0, dj0 = _start(c0, bi0, bj0, si0, sj0)
            di1, dj1 = _start(c1, bi1, bj1, si1, sj1)
            di0.wait()
            dj0.wait()
            _add(bi0, bj0)
            w0 = pltpu.async_copy(
                bi0, g_hbm.at[pl.ds(base + c0 * KG, KG), :], ws0)
            di1.wait()
            dj1.wait()
            _add(bi1, bj1)
            w1 = pltpu.async_copy(
                bi1, g_hbm.at[pl.ds(base + c1 * KG, KG), :], ws1)
            w0.wait()
            w1.wait()
            return carry

        lax.fori_loop(0, npair, pair_body, 0)
        for k in range(2 * npair, nchunk):
            di0, dj0 = _start(k, bi0, bj0, si0, sj0)
            di0.wait()
            dj0.wait()
            _add(bi0, bj0)
            pltpu.sync_copy(bi0, g_hbm.at[pl.ds(base + k * KG, KG), :])

    return gather_k


@functools.lru_cache(maxsize=None)
def _make_scatter(E, N, C, CE):
    CS = C // _NC
    ept = E // _NS
    nchunk = ept // CE
    rpt = (N // _NS) // 8 * 8
    tail = N - rpt * _NS
    mesh = plsc.VectorSubcoreMesh(core_axis_name="c", subcore_axis_name="s")

    @functools.partial(
        pl.kernel,
        out_type=jax.ShapeDtypeStruct((N, C), jnp.float32),
        mesh=mesh,
        scratch_types=[
            pltpu.VMEM_SHARED((N, CS), jnp.float32),
            pltpu.VMEM((CE, CS), jnp.float32),
            pltpu.VMEM((CE, CS), jnp.float32),
            pltpu.VMEM((CE,), jnp.int32),
            pltpu.VMEM((CE,), jnp.int32),
            pltpu.SemaphoreType.DMA,
            pltpu.SemaphoreType.DMA,
            pltpu.SemaphoreType.DMA,
            pltpu.SemaphoreType.DMA,
            pltpu.SemaphoreType.DMA,
            pltpu.SemaphoreType.DMA,
        ],
    )
    def scatter_k(m_hbm, dst_hbm, x_hbm, out_hbm, acc_sh,
                  b0, b1, i0, i1, lb0, lb1, li0, li1, ss0, ss1):
        c = lax.axis_index("c")
        s = lax.axis_index("s")
        col0 = c * CS
        r0 = s * rpt
        pltpu.sync_copy(
            x_hbm.at[pl.ds(r0, rpt), pl.ds(col0, CS)],
            acc_sh.at[pl.ds(r0, rpt), :],
        )
        if tail:
            @pl.when(s == _NS - 1)
            def _init_tail():
                pltpu.sync_copy(
                    x_hbm.at[pl.ds(rpt * _NS, tail), pl.ds(col0, CS)],
                    acc_sh.at[pl.ds(rpt * _NS, tail), :],
                )
        plsc.subcore_barrier()

        def _load(k, buf, idxb, lb, li):
            e0 = s * ept + k * CE
            return (
                pltpu.async_copy(
                    m_hbm.at[pl.ds(e0, CE), pl.ds(col0, CS)], buf, lb),
                pltpu.async_copy(dst_hbm.at[pl.ds(e0, CE)], idxb, li),
            )

        npair = nchunk // 2

        def pair_body(p, carry):
            k0 = 2 * p
            k1 = k0 + 1
            dm0, dI0 = _load(k0, b0, i0, lb0, li0)
            dm1, dI1 = _load(k1, b1, i1, lb1, li1)
            dm0.wait()
            dI0.wait()
            sc0 = pltpu.async_copy(b0, acc_sh.at[i0], ss0, add=True)
            dm1.wait()
            dI1.wait()
            sc1 = pltpu.async_copy(b1, acc_sh.at[i1], ss1, add=True)
            sc0.wait()
            sc1.wait()
            return carry

        lax.fori_loop(0, npair, pair_body, 0)
        for k in range(2 * npair, nchunk):
            dm0, dI0 = _load(k, b0, i0, lb0, li0)
            dm0.wait()
            dI0.wait()
            pltpu.sync_copy(b0, acc_sh.at[i0], add=True)
        plsc.subcore_barrier()
        pltpu.sync_copy(
            acc_sh.at[pl.ds(r0, rpt), :],
            out_hbm.at[pl.ds(r0, rpt), pl.ds(col0, CS)],
        )
        if tail:
            @pl.when(s == _NS - 1)
            def _write_tail():
                pltpu.sync_copy(
                    acc_sh.at[pl.ds(rpt * _NS, tail), :],
                    out_hbm.at[pl.ds(rpt * _NS, tail), pl.ds(col0, CS)],
                )

    return scatter_k


def kernel(x, edge_index, edge_attr, W_f, b_f, W_s, b_s, W1, b1, W2, b2):
    N, C = x.shape
    E, D_E = edge_attr.shape
    H = 2 * C

    src = edge_index[0].astype(jnp.int32)
    dst = edge_index[1].astype(jnp.int32)
    W_i = jnp.concatenate([W_f[:C], W_s[:C]], axis=1)
    W_j = jnp.concatenate([W_f[C:2 * C], W_s[C:2 * C]], axis=1)
    W_e = jnp.concatenate([W_f[2 * C:], W_s[2 * C:]], axis=1)
    b_cat = jnp.concatenate([b_f, b_s])[None, :]

    BN = 1000
    t_i, t_j = pl.pallas_call(
        _proj_body,
        grid=(N // BN,),
        in_specs=[
            pl.BlockSpec((BN, C), lambda i: (i, 0)),
            pl.BlockSpec((C, H), lambda i: (0, 0)),
            pl.BlockSpec((C, H), lambda i: (0, 0)),
            pl.BlockSpec((1, H), lambda i: (0, 0)),
        ],
        out_specs=[pl.BlockSpec((BN, H), lambda i: (i, 0))] * 2,
        out_shape=[jax.ShapeDtypeStruct((N, H), jnp.float32)] * 2,
    )(x, W_i, W_j, b_cat)

    g = _make_gather(E, H, 40)(t_i, t_j, dst, src)

    BE = 2000
    m = pl.pallas_call(
        _edge_body,
        grid=(E // BE,),
        in_specs=[
            pl.BlockSpec((BE, H), lambda i: (i, 0)),
            pl.BlockSpec((BE, D_E), lambda i: (i, 0)),
            pl.BlockSpec((D_E, H), lambda i: (0, 0)),
        ],
        out_specs=pl.BlockSpec((BE, C), lambda i: (i, 0)),
        out_shape=jax.ShapeDtypeStruct((E, C), jnp.float32),
    )(g, edge_attr, W_e)

    onode = _make_scatter(E, N, C, 80)(m, dst, x)

    y = pl.pallas_call(
        _mlp_body,
        grid=(N // BN,),
        in_specs=[
            pl.BlockSpec((BN, C), lambda i: (i, 0)),
            pl.BlockSpec((C, C), lambda i: (0, 0)),
            pl.BlockSpec((1, C), lambda i: (0, 0)),
            pl.BlockSpec((C, C), lambda i: (0, 0)),
            pl.BlockSpec((1, C), lambda i: (0, 0)),
        ],
        out_specs=pl.BlockSpec((BN, C), lambda i: (i, 0)),
        out_shape=jax.ShapeDtypeStruct((N, C), jnp.float32),
    )(onode, W1, b1[None], W2, b2[None])
    return y


# speedup vs baseline: 3.0200x; 1.3395x over previous
import functools

import jax
import jax.numpy as jnp
from jax import lax
from jax.experimental import pallas as pl
from jax.experimental.pallas import tpu as pltpu
from jax.experimental.pallas import tpu_sc as plsc

_NC = 2
_NS = 16
_LANES = 16


def _pack_bf16(v):
    c = v.shape[1] // 2
    vi = lax.bitcast_convert_type(v, jnp.int32)
    r = lax.shift_right_arithmetic(
        vi + 0x7FFF + (lax.shift_right_arithmetic(vi, 16) & 1), 16
    )
    word = (r[:, :c] & 0xFFFF) | lax.shift_left(r[:, c:], 16)
    return lax.bitcast_convert_type(word, jnp.float32)


def _unpack_bf16(w):
    wi = lax.bitcast_convert_type(w, jnp.int32)
    lo = lax.bitcast_convert_type(lax.shift_left(wi, 16), jnp.float32)
    hi = lax.bitcast_convert_type(
        wi & jnp.int32(-0x10000), jnp.float32)
    return lo, hi


def _proj_body(x_ref, wi_ref, wj_ref, bi_ref, ti_ref, tj_ref):
    xb = x_ref[...]
    ti_ref[...] = _pack_bf16(
        jnp.dot(xb, wi_ref[...], preferred_element_type=jnp.float32)
        + bi_ref[...]
    )
    tj_ref[...] = _pack_bf16(
        jnp.dot(xb, wj_ref[...], preferred_element_type=jnp.float32)
    )


def _edge_body(gi_ref, gj_ref, ea_ref, we_ref, m_ref):
    c = m_ref.shape[1]
    lo_i, hi_i = _unpack_bf16(gi_ref[...])
    lo_j, hi_j = _unpack_bf16(gj_ref[...])
    ea = jnp.dot(ea_ref[...], we_ref[...],
                 preferred_element_type=jnp.float32)
    a = lo_i + lo_j + ea[:, :c]
    s = hi_i + hi_j + ea[:, c:]
    sig = 1.0 / (1.0 + jnp.exp(-a))
    sp = jnp.maximum(s, 0.0) + jnp.log(1.0 + jnp.exp(-jnp.abs(s)))
    m_ref[...] = sig * sp


def _mlp_body(o_ref, w1_ref, b1_ref, w2_ref, b2_ref, y_ref):
    h = jnp.maximum(
        jnp.dot(o_ref[...], w1_ref[...], preferred_element_type=jnp.float32)
        + b1_ref[...],
        0.0,
    )
    y_ref[...] = (
        jnp.dot(h, w2_ref[...], preferred_element_type=jnp.float32)
        + b2_ref[...]
    )


@functools.lru_cache(maxsize=None)
def _make_gather(E, HP, KG, NB=4):
    NW = _NC * _NS
    epw = E // NW
    nchunk = epw // KG
    ngrp = nchunk // NB
    mesh = plsc.VectorSubcoreMesh(core_axis_name="c", subcore_axis_name="s")
    out_sd = jax.ShapeDtypeStruct((E, HP), jnp.float32)

    @functools.partial(
        pl.kernel,
        out_type=(out_sd, out_sd),
        mesh=mesh,
        scratch_types=[
            pltpu.VMEM((epw,), jnp.int32),
            pltpu.VMEM((epw,), jnp.int32),
        ]
        + [pltpu.VMEM((KG, HP), jnp.float32)] * (2 * NB)
        + [pltpu.SemaphoreType.DMA] * (2 * NB),
    )
    def gather_k(ti_hbm, tj_hbm, dst_hbm, src_hbm, gi_hbm, gj_hbm,
                 idx_d, idx_s, *bufs_and_sems):
        bi = bufs_and_sems[0:NB]
        bj = bufs_and_sems[NB:2 * NB]
        sg = bufs_and_sems[2 * NB:3 * NB]
        sw = bufs_and_sems[3 * NB:4 * NB]
        wid = lax.axis_index("s") * _NC + lax.axis_index("c")
        base = wid * epw
        pltpu.sync_copy(dst_hbm.at[pl.ds(base, epw)], idx_d)
        pltpu.sync_copy(src_hbm.at[pl.ds(base, epw)], idx_s)

        def _gathers(k, b):
            sl = pl.ds(k * KG, KG)
            return (
                pltpu.async_copy(ti_hbm.at[idx_d.at[sl]], bi[b], sg[b]),
                pltpu.async_copy(tj_hbm.at[idx_s.at[sl]], bj[b], sg[b]),
            )

        def _writes(k, b):
            e0 = base + k * KG
            return (
                pltpu.async_copy(bi[b], gi_hbm.at[pl.ds(e0, KG), :], sw[b]),
                pltpu.async_copy(bj[b], gj_hbm.at[pl.ds(e0, KG), :], sw[b]),
            )

        def grp_body(g, carry):
            k0 = g * NB
            gd = [_gathers(k0 + b, b) for b in range(NB)]
            wd = []
            for b in range(NB):
                gd[b][0].wait()
                gd[b][1].wait()
                wd.append(_writes(k0 + b, b))
            for b in range(NB):
                wd[b][0].wait()
                wd[b][1].wait()
            return carry

        lax.fori_loop(0, ngrp, grp_body, 0)
        for k in range(ngrp * NB, nchunk):
            di, dj = _gathers(k, 0)
            di.wait()
            dj.wait()
            wi, wj = _writes(k, 0)
            wi.wait()
            wj.wait()

    return gather_k


@functools.lru_cache(maxsize=None)
def _make_scatter(E, N, C, CE):
    CS = C // _NC
    ept = E // _NS
    nchunk = ept // CE
    rpt = (N // _NS) // 8 * 8
    tail = N - rpt * _NS
    mesh = plsc.VectorSubcoreMesh(core_axis_name="c", subcore_axis_name="s")

    @functools.partial(
        pl.kernel,
        out_type=jax.ShapeDtypeStruct((N, C), jnp.float32),
        mesh=mesh,
        scratch_types=[
            pltpu.VMEM_SHARED((N, CS), jnp.float32),
            pltpu.VMEM((CE, CS), jnp.float32),
            pltpu.VMEM((CE, CS), jnp.float32),
            pltpu.VMEM((CE,), jnp.int32),
            pltpu.VMEM((CE,), jnp.int32),
            pltpu.SemaphoreType.DMA,
            pltpu.SemaphoreType.DMA,
            pltpu.SemaphoreType.DMA,
            pltpu.SemaphoreType.DMA,
            pltpu.SemaphoreType.DMA,
            pltpu.SemaphoreType.DMA,
        ],
    )
    def scatter_k(m_hbm, dst_hbm, x_hbm, out_hbm, acc_sh,
                  b0, b1, i0, i1, lb0, lb1, li0, li1, ss0, ss1):
        c = lax.axis_index("c")
        s = lax.axis_index("s")
        col0 = c * CS
        r0 = s * rpt
        pltpu.sync_copy(
            x_hbm.at[pl.ds(r0, rpt), pl.ds(col0, CS)],
            acc_sh.at[pl.ds(r0, rpt), :],
        )
        if tail:
            @pl.when(s == _NS - 1)
            def _init_tail():
                pltpu.sync_copy(
                    x_hbm.at[pl.ds(rpt * _NS, tail), pl.ds(col0, CS)],
                    acc_sh.at[pl.ds(rpt * _NS, tail), :],
                )
        plsc.subcore_barrier()

        def _load(k, buf, idxb, lb, li):
            e0 = s * ept + k * CE
            return (
                pltpu.async_copy(
                    m_hbm.at[pl.ds(e0, CE), pl.ds(col0, CS)], buf, lb),
                pltpu.async_copy(dst_hbm.at[pl.ds(e0, CE)], idxb, li),
            )

        npair = nchunk // 2

        def pair_body(p, carry):
            k0 = 2 * p
            k1 = k0 + 1
            dm0, dI0 = _load(k0, b0, i0, lb0, li0)
            dm1, dI1 = _load(k1, b1, i1, lb1, li1)
            dm0.wait()
            dI0.wait()
            sc0 = pltpu.async_copy(b0, acc_sh.at[i0], ss0, add=True)
            dm1.wait()
            dI1.wait()
            sc1 = pltpu.async_copy(b1, acc_sh.at[i1], ss1, add=True)
            sc0.wait()
            sc1.wait()
            return carry

        lax.fori_loop(0, npair, pair_body, 0)
        for k in range(2 * npair, nchunk):
            dm0, dI0 = _load(k, b0, i0, lb0, li0)
            dm0.wait()
            dI0.wait()
            pltpu.sync_copy(b0, acc_sh.at[i0], add=True)
        plsc.subcore_barrier()
        pltpu.sync_copy(
            acc_sh.at[pl.ds(r0, rpt), :],
            out_hbm.at[pl.ds(r0, rpt), pl.ds(col0, CS)],
        )
        if tail:
            @pl.when(s == _NS - 1)
            def _write_tail():
                pltpu.sync_copy(
                    acc_sh.at[pl.ds(rpt * _NS, tail), :],
                    out_hbm.at[pl.ds(rpt * _NS, tail), pl.ds(col0, CS)],
                )

    return scatter_k


def kernel(x, edge_index, edge_attr, W_f, b_f, W_s, b_s, W1, b1, W2, b2):
    N, C = x.shape
    E, D_E = edge_attr.shape
    H = 2 * C

    src = edge_index[0].astype(jnp.int32)
    dst = edge_index[1].astype(jnp.int32)
    W_i = jnp.concatenate([W_f[:C], W_s[:C]], axis=1)
    W_j = jnp.concatenate([W_f[C:2 * C], W_s[C:2 * C]], axis=1)
    W_e = jnp.concatenate([W_f[2 * C:], W_s[2 * C:]], axis=1)
    b_cat = jnp.concatenate([b_f, b_s])[None, :]

    BN = 1000
    t_i, t_j = pl.pallas_call(
        _proj_body,
        grid=(N // BN,),
        in_specs=[
            pl.BlockSpec((BN, C), lambda i: (i, 0)),
            pl.BlockSpec((C, H), lambda i: (0, 0)),
            pl.BlockSpec((C, H), lambda i: (0, 0)),
            pl.BlockSpec((1, H), lambda i: (0, 0)),
        ],
        out_specs=[pl.BlockSpec((BN, C), lambda i: (i, 0))] * 2,
        out_shape=[jax.ShapeDtypeStruct((N, C), jnp.float32)] * 2,
    )(x, W_i, W_j, b_cat)

    g_i, g_j = _make_gather(E, C, 40)(t_i, t_j, dst, src)

    BE = 2000
    m = pl.pallas_call(
        _edge_body,
        grid=(E // BE,),
        in_specs=[
            pl.BlockSpec((BE, C), lambda i: (i, 0)),
            pl.BlockSpec((BE, C), lambda i: (i, 0)),
            pl.BlockSpec((BE, D_E), lambda i: (i, 0)),
            pl.BlockSpec((D_E, H), lambda i: (0, 0)),
        ],
        out_specs=pl.BlockSpec((BE, C), lambda i: (i, 0)),
        out_shape=jax.ShapeDtypeStruct((E, C), jnp.float32),
    )(g_i, g_j, edge_attr, W_e)

    onode = _make_scatter(E, N, C, 80)(m, dst, x)

    y = pl.pallas_call(
        _mlp_body,
        grid=(N // BN,),
        in_specs=[
            pl.BlockSpec((BN, C), lambda i: (i, 0)),
            pl.BlockSpec((C, C), lambda i: (0, 0)),
            pl.BlockSpec((1, C), lambda i: (0, 0)),
            pl.BlockSpec((C, C), lambda i: (0, 0)),
            pl.BlockSpec((1, C), lambda i: (0, 0)),
        ],
        out_specs=pl.BlockSpec((BN, C), lambda i: (i, 0)),
        out_shape=jax.ShapeDtypeStruct((N, C), jnp.float32),
    )(onode, W1, b1[None], W2, b2[None])
    return y


# speedup vs baseline: 3.2338x; 1.0708x over previous
import functools

import jax
import jax.numpy as jnp
from jax import lax
from jax.experimental import pallas as pl
from jax.experimental.pallas import tpu as pltpu
from jax.experimental.pallas import tpu_sc as plsc

_NC = 2
_NS = 16
_LANES = 16


def _pack_bf16(v):
    c = v.shape[1] // 2
    vi = lax.bitcast_convert_type(v, jnp.int32)
    r = lax.shift_right_arithmetic(
        vi + 0x7FFF + (lax.shift_right_arithmetic(vi, 16) & 1), 16
    )
    word = (r[:, :c] & 0xFFFF) | lax.shift_left(r[:, c:], 16)
    return lax.bitcast_convert_type(word, jnp.float32)


def _unpack_bf16(w):
    wi = lax.bitcast_convert_type(w, jnp.int32)
    lo = lax.bitcast_convert_type(lax.shift_left(wi, 16), jnp.float32)
    hi = lax.bitcast_convert_type(
        wi & jnp.int32(-0x10000), jnp.float32)
    return lo, hi


def _proj_body(x_ref, wi_ref, wj_ref, bi_ref, ti_ref, tj_ref):
    xb = x_ref[...]
    ti_ref[...] = _pack_bf16(
        jnp.dot(xb, wi_ref[...], preferred_element_type=jnp.float32)
        + bi_ref[...]
    )
    tj_ref[...] = _pack_bf16(
        jnp.dot(xb, wj_ref[...], preferred_element_type=jnp.float32)
    )


def _edge_body(gi_ref, gj_ref, ea_ref, we_ref, m_ref):
    c = m_ref.shape[1]
    lo_i, hi_i = _unpack_bf16(gi_ref[...])
    lo_j, hi_j = _unpack_bf16(gj_ref[...])
    ea = jnp.dot(ea_ref[...], we_ref[...],
                 preferred_element_type=jnp.float32)
    a = lo_i + lo_j + ea[:, :c]
    s = hi_i + hi_j + ea[:, c:]
    sig = 1.0 / (1.0 + jnp.exp(-a))
    sp = jnp.maximum(s, 0.0) + jnp.log(1.0 + jnp.exp(-jnp.abs(s)))
    m_ref[...] = sig * sp


def _mlp_body(o_ref, w1_ref, b1_ref, w2_ref, b2_ref, y_ref):
    h = jnp.maximum(
        jnp.dot(o_ref[...], w1_ref[...], preferred_element_type=jnp.float32)
        + b1_ref[...],
        0.0,
    )
    y_ref[...] = (
        jnp.dot(h, w2_ref[...], preferred_element_type=jnp.float32)
        + b2_ref[...]
    )


@functools.lru_cache(maxsize=None)
def _make_gather(E, HP, KG, NB=4):
    NW = _NC * _NS
    epw = E // NW
    nchunk = epw // KG
    ngrp = nchunk // NB
    mesh = plsc.VectorSubcoreMesh(core_axis_name="c", subcore_axis_name="s")
    out_sd = jax.ShapeDtypeStruct((E, HP), jnp.float32)

    @functools.partial(
        pl.kernel,
        out_type=(out_sd, out_sd),
        mesh=mesh,
        scratch_types=[
            pltpu.VMEM((epw,), jnp.int32),
            pltpu.VMEM((epw,), jnp.int32),
        ]
        + [pltpu.VMEM((KG, HP), jnp.float32)] * (2 * NB)
        + [pltpu.SemaphoreType.DMA] * (2 * NB),
    )
    def gather_k(ti_hbm, tj_hbm, dst_hbm, src_hbm, gi_hbm, gj_hbm,
                 idx_d, idx_s, *bufs_and_sems):
        bi = bufs_and_sems[0:NB]
        bj = bufs_and_sems[NB:2 * NB]
        sg = bufs_and_sems[2 * NB:3 * NB]
        sw = bufs_and_sems[3 * NB:4 * NB]
        wid = lax.axis_index("s") * _NC + lax.axis_index("c")
        base = wid * epw
        pltpu.sync_copy(dst_hbm.at[pl.ds(base, epw)], idx_d)
        pltpu.sync_copy(src_hbm.at[pl.ds(base, epw)], idx_s)

        def _gathers(k, b):
            sl = pl.ds(k * KG, KG)
            return (
                pltpu.async_copy(ti_hbm.at[idx_d.at[sl]], bi[b], sg[b]),
                pltpu.async_copy(tj_hbm.at[idx_s.at[sl]], bj[b], sg[b]),
            )

        def _writes(k, b):
            e0 = base + k * KG
            return (
                pltpu.async_copy(bi[b], gi_hbm.at[pl.ds(e0, KG), :], sw[b]),
                pltpu.async_copy(bj[b], gj_hbm.at[pl.ds(e0, KG), :], sw[b]),
            )

        def grp_body(g, carry):
            k0 = g * NB
            gd = [_gathers(k0 + b, b) for b in range(NB)]
            wd = []
            for b in range(NB):
                gd[b][0].wait()
                gd[b][1].wait()
                wd.append(_writes(k0 + b, b))
            for b in range(NB):
                wd[b][0].wait()
                wd[b][1].wait()
            return carry

        lax.fori_loop(0, ngrp, grp_body, 0)
        for k in range(ngrp * NB, nchunk):
            di, dj = _gathers(k, 0)
            di.wait()
            dj.wait()
            wi, wj = _writes(k, 0)
            wi.wait()
            wj.wait()

    return gather_k


@functools.lru_cache(maxsize=None)
def _make_scatter(E, N, C, CE):
    CS = C // _NC
    ept = E // _NS
    nchunk = ept // CE
    rpt = (N // _NS) // 8 * 8
    tail = N - rpt * _NS
    mesh = plsc.VectorSubcoreMesh(core_axis_name="c", subcore_axis_name="s")

    @functools.partial(
        pl.kernel,
        out_type=jax.ShapeDtypeStruct((N, C), jnp.float32),
        mesh=mesh,
        scratch_types=[
            pltpu.VMEM_SHARED((N, CS), jnp.float32),
            pltpu.VMEM((CE, CS), jnp.float32),
            pltpu.VMEM((CE, CS), jnp.float32),
            pltpu.VMEM((CE,), jnp.int32),
            pltpu.VMEM((CE,), jnp.int32),
            pltpu.SemaphoreType.DMA,
            pltpu.SemaphoreType.DMA,
            pltpu.SemaphoreType.DMA,
            pltpu.SemaphoreType.DMA,
            pltpu.SemaphoreType.DMA,
            pltpu.SemaphoreType.DMA,
        ],
    )
    def scatter_k(m_hbm, dst_hbm, x_hbm, out_hbm, acc_sh,
                  b0, b1, i0, i1, lb0, lb1, li0, li1, ss0, ss1):
        c = lax.axis_index("c")
        s = lax.axis_index("s")
        col0 = c * CS
        r0 = s * rpt
        pltpu.sync_copy(
            x_hbm.at[pl.ds(r0, rpt), pl.ds(col0, CS)],
            acc_sh.at[pl.ds(r0, rpt), :],
        )
        if tail:
            @pl.when(s == _NS - 1)
            def _init_tail():
                pltpu.sync_copy(
                    x_hbm.at[pl.ds(rpt * _NS, tail), pl.ds(col0, CS)],
                    acc_sh.at[pl.ds(rpt * _NS, tail), :],
                )
        plsc.subcore_barrier()

        def _load(k, buf, idxb, lb, li):
            e0 = s * ept + k * CE
            return (
                pltpu.async_copy(
                    m_hbm.at[pl.ds(e0, CE), pl.ds(col0, CS)], buf, lb),
                pltpu.async_copy(dst_hbm.at[pl.ds(e0, CE)], idxb, li),
            )

        npair = nchunk // 2

        def pair_body(p, carry):
            k0 = 2 * p
            k1 = k0 + 1
            dm0, dI0 = _load(k0, b0, i0, lb0, li0)
            dm1, dI1 = _load(k1, b1, i1, lb1, li1)
            dm0.wait()
            dI0.wait()
            sc0 = pltpu.async_copy(b0, acc_sh.at[i0], ss0, add=True)
            dm1.wait()
            dI1.wait()
            sc1 = pltpu.async_copy(b1, acc_sh.at[i1], ss1, add=True)
            sc0.wait()
            sc1.wait()
            return carry

        lax.fori_loop(0, npair, pair_body, 0)
        for k in range(2 * npair, nchunk):
            dm0, dI0 = _load(k, b0, i0, lb0, li0)
            dm0.wait()
            dI0.wait()
            pltpu.sync_copy(b0, acc_sh.at[i0], add=True)
        plsc.subcore_barrier()
        pltpu.sync_copy(
            acc_sh.at[pl.ds(r0, rpt), :],
            out_hbm.at[pl.ds(r0, rpt), pl.ds(col0, CS)],
        )
        if tail:
            @pl.when(s == _NS - 1)
            def _write_tail():
                pltpu.sync_copy(
                    acc_sh.at[pl.ds(rpt * _NS, tail), :],
                    out_hbm.at[pl.ds(rpt * _NS, tail), pl.ds(col0, CS)],
                )

    return scatter_k


def kernel(x, edge_index, edge_attr, W_f, b_f, W_s, b_s, W1, b1, W2, b2):
    N, C = x.shape
    E, D_E = edge_attr.shape
    H = 2 * C

    src = edge_index[0].astype(jnp.int32)
    dst = edge_index[1].astype(jnp.int32)
    W_i = jnp.concatenate([W_f[:C], W_s[:C]], axis=1)
    W_j = jnp.concatenate([W_f[C:2 * C], W_s[C:2 * C]], axis=1)
    W_e = jnp.concatenate([W_f[2 * C:], W_s[2 * C:]], axis=1)
    b_cat = jnp.concatenate([b_f, b_s])[None, :]

    BN = 1000
    t_i, t_j = pl.pallas_call(
        _proj_body,
        grid=(N // BN,),
        in_specs=[
            pl.BlockSpec((BN, C), lambda i: (i, 0)),
            pl.BlockSpec((C, H), lambda i: (0, 0)),
            pl.BlockSpec((C, H), lambda i: (0, 0)),
            pl.BlockSpec((1, H), lambda i: (0, 0)),
        ],
        out_specs=[pl.BlockSpec((BN, C), lambda i: (i, 0))] * 2,
        out_shape=[jax.ShapeDtypeStruct((N, C), jnp.float32)] * 2,
    )(x, W_i, W_j, b_cat)

    BE = 2000
    splits = [(0, 2 * E // 5), (2 * E // 5, E)]
    onode = x
    for lo, hi in splits:
        ep = hi - lo
        d_p = dst[lo:hi]
        g_i, g_j = _make_gather(ep, C, 40)(t_i, t_j, d_p, src[lo:hi])
        m = pl.pallas_call(
            _edge_body,
            grid=(ep // BE,),
            in_specs=[
                pl.BlockSpec((BE, C), lambda i: (i, 0)),
                pl.BlockSpec((BE, C), lambda i: (i, 0)),
                pl.BlockSpec((BE, D_E), lambda i: (i, 0)),
                pl.BlockSpec((D_E, H), lambda i: (0, 0)),
            ],
            out_specs=pl.BlockSpec((BE, C), lambda i: (i, 0)),
            out_shape=jax.ShapeDtypeStruct((ep, C), jnp.float32),
        )(g_i, g_j, edge_attr[lo:hi], W_e)
        onode = _make_scatter(ep, N, C, 80)(m, d_p, onode)

    y = pl.pallas_call(
        _mlp_body,
        grid=(N // BN,),
        in_specs=[
            pl.BlockSpec((BN, C), lambda i: (i, 0)),
            pl.BlockSpec((C, C), lambda i: (0, 0)),
            pl.BlockSpec((1, C), lambda i: (0, 0)),
            pl.BlockSpec((C, C), lambda i: (0, 0)),
            pl.BlockSpec((1, C), lambda i: (0, 0)),
        ],
        out_specs=pl.BlockSpec((BN, C), lambda i: (i, 0)),
        out_shape=jax.ShapeDtypeStruct((N, C), jnp.float32),
    )(onode, W1, b1[None], W2, b2[None])
    return y
